# baseline (speedup 1.0000x reference)
import functools

import jax
import jax.numpy as jnp
from jax import lax
from jax.experimental import pallas as pl
from jax.experimental.pallas import tpu as pltpu
from jax.experimental.pallas import tpu_sc as plsc

D = 64
CHUNK = 128
K = 5


@functools.cache
def _make_kernel(n_iters: int, nc: int, ns: int):
    nw = nc * ns
    n_blocks = nw * n_iters
    mesh = plsc.VectorSubcoreMesh(core_axis_name="c", subcore_axis_name="s")

    @functools.partial(
        pl.kernel,
        out_type=jax.ShapeDtypeStruct((n_blocks * K, CHUNK, D), jnp.float32),
        mesh=mesh,
        scratch_types=[
            pltpu.VMEM((K, CHUNK), jnp.int32),
            pltpu.VMEM((K, CHUNK, D), jnp.float32),
            pltpu.SemaphoreType.DMA,
        ],
        compiler_params=pltpu.CompilerParams(use_tc_tiling_on_sc=False),
    )
    def emb_kernel(idx_hbm, w_hbm, out_hbm, idx_v, rows_v, sem):
        wid = lax.axis_index("s") * nc + lax.axis_index("c")

        def body(i, carry):
            blk = wid * n_iters + i
            pltpu.sync_copy(idx_hbm.at[blk], idx_v)
            handles = [
                pltpu.async_copy(w_hbm.at[idx_v.at[j]], rows_v.at[j], sem)
                for j in range(K)
            ]
            for h in handles:
                h.wait()
            pltpu.sync_copy(rows_v, out_hbm.at[pl.ds(blk * K, K)])
            return carry

        lax.fori_loop(0, n_iters, body, 0)

    return emb_kernel


def kernel(token_ids, weight):
    b, h = token_ids.shape
    n = b * h
    info = plsc.get_sparse_core_info()
    nc, ns = info.num_cores, info.num_subcores
    nw = nc * ns
    rows_per_iter = K * CHUNK
    assert n % (nw * rows_per_iter) == 0
    n_iters = n // (nw * rows_per_iter)
    idx = token_ids.reshape(nw * n_iters, K, CHUNK).astype(jnp.int32)
    out = _make_kernel(n_iters, nc, ns)(idx, weight)
    return out.reshape(b, h, D)


# speedup vs baseline: 1.0345x; 1.0345x over previous
import functools

import jax
import jax.numpy as jnp
from jax import lax
from jax.experimental import pallas as pl
from jax.experimental.pallas import tpu as pltpu
from jax.experimental.pallas import tpu_sc as plsc

D = 64
CHUNK = 128
NBUF = 8
DEPTH = 4


@functools.cache
def _make_kernel(n_chunks_per_w: int, nc: int, ns: int):
    nw = nc * ns
    n_chunks = nw * n_chunks_per_w
    mesh = plsc.VectorSubcoreMesh(core_axis_name="c", subcore_axis_name="s")
    assert n_chunks_per_w % NBUF == 0 and n_chunks_per_w > 2 * NBUF

    @functools.partial(
        pl.kernel,
        out_type=jax.ShapeDtypeStruct((n_chunks, CHUNK, D), jnp.float32),
        mesh=mesh,
        scratch_types=[
            pltpu.VMEM((n_chunks_per_w, CHUNK), jnp.int32),
            pltpu.VMEM((NBUF, CHUNK, D), jnp.float32),
            pltpu.SemaphoreType.DMA((NBUF,)),
            pltpu.SemaphoreType.DMA((NBUF,)),
        ],
        compiler_params=pltpu.CompilerParams(use_tc_tiling_on_sc=False),
    )
    def emb_kernel(idx_hbm, w_hbm, out_hbm, idx_v, rows_v, gsem, osem):
        wid = lax.axis_index("s") * nc + lax.axis_index("c")
        out_base = wid * n_chunks_per_w

        pltpu.sync_copy(idx_hbm.at[wid], idx_v)

        def fire_gather(s, b):
            pltpu.async_copy(w_hbm.at[idx_v.at[s]], rows_v.at[b], gsem.at[b])

        def wait_gather(s, b):
            pltpu.make_async_copy(
                w_hbm.at[idx_v.at[s]], rows_v.at[b], gsem.at[b]).wait()

        def fire_wb(s, b):
            pltpu.async_copy(rows_v.at[b], out_hbm.at[out_base + s], osem.at[b])

        def wait_wb(s, b):
            pltpu.make_async_copy(
                rows_v.at[b], out_hbm.at[out_base + s], osem.at[b]).wait()

        for s0 in range(DEPTH):
            fire_gather(s0, s0 % NBUF)

        @pl.loop(0, n_chunks_per_w, step=NBUF)
        def _(c0):
            for b in range(NBUF):
                s = c0 + b
                wait_gather(s, b)
                fire_wb(s, b)
                bn = (b + DEPTH) % NBUF
                lag = NBUF - DEPTH

                @pl.when(s >= lag)
                def _():
                    wait_wb(s - lag, bn)

                @pl.when(s + DEPTH < n_chunks_per_w)
                def _():
                    fire_gather(s + DEPTH, bn)

        for s in range(n_chunks_per_w - (NBUF - DEPTH), n_chunks_per_w):
            wait_wb(s, s % NBUF)

    return emb_kernel


def kernel(token_ids, weight):
    b, h = token_ids.shape
    n = b * h
    info = plsc.get_sparse_core_info()
    nc, ns = info.num_cores, info.num_subcores
    nw = nc * ns
    assert n % (nw * CHUNK) == 0
    n_chunks_per_w = n // (nw * CHUNK)
    idx = token_ids.reshape(nw, n_chunks_per_w, CHUNK).astype(jnp.int32)
    out = _make_kernel(n_chunks_per_w, nc, ns)(idx, weight)
    return out.reshape(b, h, D)
